# baseline (speedup 1.0000x reference)
import functools

import jax
import jax.numpy as jnp
from jax import lax
from jax.experimental import pallas as pl
from jax.experimental.pallas import tpu as pltpu
from jax.experimental.pallas import tpu_sc as plsc

N = 10000
D = 128
E = 320000

NTILES = 32
NSUB = 16
CH = 128
CPT = 80
HALF = CPT // 2
E_PAD = NTILES * CPT * CH
N_ACC = N + 8

ROWS_PT = 624
TAIL_OFF = ROWS_PT * NSUB

BM = 1000
GRID = N // BM

_mesh = plsc.VectorSubcoreMesh(core_axis_name="c", subcore_axis_name="s")


def _acc_init(zeros_hbm, acc_sh, s):
    pltpu.sync_copy(
        zeros_hbm.at[pl.ds(s * ROWS_PT, ROWS_PT)],
        acc_sh.at[pl.ds(s * ROWS_PT, ROWS_PT)],
    )

    @pl.when(s == 0)
    def _():
        pltpu.sync_copy(
            zeros_hbm.at[pl.ds(TAIL_OFF, N_ACC - TAIL_OFF)],
            acc_sh.at[pl.ds(TAIL_OFF, N_ACC - TAIL_OFF)],
        )


def _acc_writeout(acc_sh, out_hbm, c, s):
    pltpu.sync_copy(
        acc_sh.at[pl.ds(s * ROWS_PT, ROWS_PT)],
        out_hbm.at[c, pl.ds(s * ROWS_PT, ROWS_PT)],
    )

    @pl.when(s == 0)
    def _():
        pltpu.sync_copy(
            acc_sh.at[pl.ds(TAIL_OFF, N - TAIL_OFF)],
            out_hbm.at[c, pl.ds(TAIL_OFF, N - TAIL_OFF)],
        )


EPT = E // NTILES
HCOL = 8
NVEC = EPT // 16


def _sc_degree(dst_flat, zeros_hist):
    @functools.partial(
        pl.kernel,
        out_type=jax.ShapeDtypeStruct((NTILES * HCOL * N,), jnp.float32),
        mesh=_mesh,
        scratch_types=[
            pltpu.VMEM((EPT,), jnp.int32),
            pltpu.VMEM((HCOL * N,), jnp.float32),
        ],
        compiler_params=pltpu.CompilerParams(needs_layout_passes=False),
    )
    def deg_kernel(dst_hbm, zeros_hbm, out_hbm, dst_t, hist):
        c = lax.axis_index("c")
        s = lax.axis_index("s")
        t = c * NSUB + s
        pltpu.sync_copy(zeros_hbm, hist)
        pltpu.sync_copy(dst_hbm.at[t], dst_t)

        lane = lax.iota(jnp.int32, 16)
        base = lax.bitwise_and(lane, 7) * N
        mlo = lane < 8
        mhi = lane >= 8
        ones = jnp.full((16,), 1.0, jnp.float32)

        @pl.loop(0, NVEC)
        def _(i):
            addr = base + dst_t[pl.ds(i * 16, 16)]
            plsc.addupdate_scatter(hist, [addr], ones, mask=mlo)
            plsc.addupdate_scatter(hist, [addr], ones, mask=mhi)

        pltpu.sync_copy(hist, out_hbm.at[pl.ds(t * HCOL * N, HCOL * N)])

    return deg_kernel(dst_flat, zeros_hist)


def _sc_gather_scatter(hp, src2, dst2, zeros_acc):
    @functools.partial(
        pl.kernel,
        out_type=jax.ShapeDtypeStruct((2, N, D), jnp.float32),
        mesh=_mesh,
        scratch_types=[
            pltpu.VMEM((HALF, CH), jnp.int32),
            pltpu.VMEM((HALF, CH), jnp.int32),
            pltpu.VMEM((CH, D), jnp.float32),
            pltpu.VMEM((CH, D), jnp.float32),
            pltpu.VMEM_SHARED((N_ACC, D), jnp.float32),
            pltpu.SemaphoreType.DMA,
            pltpu.SemaphoreType.DMA,
            pltpu.SemaphoreType.DMA,
            pltpu.SemaphoreType.DMA,
        ],
    )
    def gs_kernel(hp_hbm, src_hbm, dst_hbm, zeros_hbm, out_hbm,
                  src_t, dst_t, rows0, rows1, acc_sh, sg0, sg1, ss0, ss1):
        c = lax.axis_index("c")
        s = lax.axis_index("s")
        t = c * NSUB + s
        _acc_init(zeros_hbm, acc_sh, s)
        plsc.subcore_barrier()

        for h in range(CPT // HALF):
            pltpu.sync_copy(src_hbm.at[t, pl.ds(h * HALF, HALF)], src_t)
            pltpu.sync_copy(dst_hbm.at[t, pl.ds(h * HALF, HALF)], dst_t)

            @pl.loop(0, HALF // 2)
            def _(g):
                i0 = 2 * g
                i1 = 2 * g + 1

                @pl.when(g > 0)
                def _():
                    pltpu.make_async_copy(rows0, acc_sh.at[dst_t.at[i0]], ss0).wait()
                    pltpu.make_async_copy(rows1, acc_sh.at[dst_t.at[i1]], ss1).wait()

                g0 = pltpu.async_copy(hp_hbm.at[src_t.at[i0]], rows0, sg0)
                g1 = pltpu.async_copy(hp_hbm.at[src_t.at[i1]], rows1, sg1)
                g0.wait()
                pltpu.async_copy(rows0, acc_sh.at[dst_t.at[i0]], ss0, add=True)
                g1.wait()
                pltpu.async_copy(rows1, acc_sh.at[dst_t.at[i1]], ss1, add=True)

            pltpu.make_async_copy(rows0, acc_sh.at[dst_t.at[0]], ss0).wait()
            pltpu.make_async_copy(rows1, acc_sh.at[dst_t.at[0]], ss1).wait()

        plsc.subcore_barrier()
        _acc_writeout(acc_sh, out_hbm, c, s)

    return gs_kernel(hp, src2, dst2, zeros_acc)


_DN = (((1,), (0,)), ((), ()))


def _matmul(a_ref, w_ref):
    return lax.dot_general(a_ref, w_ref, _DN,
                           preferred_element_type=jnp.float32,
                           precision=lax.Precision.HIGHEST)


def _tc_fold(degp):
    def body(deg_ref, out_ref):
        out_ref[...] = lax.rsqrt(
            jnp.sum(deg_ref[...], axis=0, keepdims=True) + 1.0
        )

    return pl.pallas_call(
        body,
        in_specs=[pl.BlockSpec((NTILES * HCOL, N), lambda: (0, 0))],
        out_specs=pl.BlockSpec((1, N), lambda: (0, 0)),
        out_shape=jax.ShapeDtypeStruct((1, N), jnp.float32),
    )(degp)


def _tc_mm1(x, W1):
    def body(x_ref, w_ref, out_ref):
        out_ref[...] = _matmul(x_ref[...], w_ref[...])

    return pl.pallas_call(
        body,
        grid=(GRID,),
        in_specs=[
            pl.BlockSpec((BM, D), lambda i: (i, 0)),
            pl.BlockSpec((D, D), lambda i: (0, 0)),
        ],
        out_specs=pl.BlockSpec((BM, D), lambda i: (i, 0)),
        out_shape=jax.ShapeDtypeStruct((N, D), jnp.float32),
    )(x, W1)


def _tc_scale1(xw1, dinvc):
    def body(xw_ref, dinv_ref, hp_ref, dinvb_ref):
        dinv = jnp.broadcast_to(dinv_ref[...], (BM, D))
        hp_ref[...] = xw_ref[...] * dinv
        dinvb_ref[...] = dinv

    return pl.pallas_call(
        body,
        grid=(GRID,),
        in_specs=[
            pl.BlockSpec((BM, D), lambda i: (i, 0)),
            pl.BlockSpec((BM, 1), lambda i: (i, 0)),
        ],
        out_specs=[
            pl.BlockSpec((BM, D), lambda i: (i, 0)),
            pl.BlockSpec((BM, D), lambda i: (i, 0)),
        ],
        out_shape=[
            jax.ShapeDtypeStruct((N, D), jnp.float32),
            jax.ShapeDtypeStruct((N, D), jnp.float32),
        ],
    )(xw1, dinvc)


def _tc_mid(agg, hp, dinv, b1, W2):
    def body(agg_ref, hp_ref, dinv_ref, b_ref, w_ref, out_ref):
        total = agg_ref[0] + agg_ref[1] + hp_ref[...]
        out1 = jnp.maximum(dinv_ref[...] * total + b_ref[...], 0.0)
        out_ref[...] = dinv_ref[...] * _matmul(out1, w_ref[...])

    return pl.pallas_call(
        body,
        grid=(GRID,),
        in_specs=[
            pl.BlockSpec((2, BM, D), lambda i: (0, i, 0)),
            pl.BlockSpec((BM, D), lambda i: (i, 0)),
            pl.BlockSpec((BM, D), lambda i: (i, 0)),
            pl.BlockSpec((1, D), lambda i: (0, 0)),
            pl.BlockSpec((D, D), lambda i: (0, 0)),
        ],
        out_specs=pl.BlockSpec((BM, D), lambda i: (i, 0)),
        out_shape=jax.ShapeDtypeStruct((N, D), jnp.float32),
    )(agg, hp, dinv, b1, W2)


def _tc_last(agg, hp, dinv, b2, W_out, b_out):
    def body(agg_ref, hp_ref, dinv_ref, b2_ref, w_ref, bo_ref, out_ref):
        total = agg_ref[0] + agg_ref[1] + hp_ref[...]
        out2 = dinv_ref[...] * total + b2_ref[...]
        out_ref[...] = _matmul(out2, w_ref[...]) + bo_ref[...]

    return pl.pallas_call(
        body,
        grid=(GRID,),
        in_specs=[
            pl.BlockSpec((2, BM, D), lambda i: (0, i, 0)),
            pl.BlockSpec((BM, D), lambda i: (i, 0)),
            pl.BlockSpec((BM, D), lambda i: (i, 0)),
            pl.BlockSpec((1, D), lambda i: (0, 0)),
            pl.BlockSpec((D, D), lambda i: (0, 0)),
            pl.BlockSpec((1, D), lambda i: (0, 0)),
        ],
        out_specs=pl.BlockSpec((BM, D), lambda i: (i, 0)),
        out_shape=jax.ShapeDtypeStruct((N, D), jnp.float32),
    )(agg, hp, dinv, b2, W_out, b_out)


def kernel(x, edge_index, W1, b1, W2, b2, W_out, b_out):
    ppt = CPT * CH - E // NTILES
    pad_src = jnp.broadcast_to(jnp.arange(ppt, dtype=jnp.int32), (NTILES, ppt))
    pad_dst = jnp.full((NTILES, ppt), N, jnp.int32)
    src2 = jnp.concatenate(
        [edge_index[0].reshape(NTILES, E // NTILES), pad_src], axis=1
    ).reshape(NTILES, CPT, CH)
    dst2 = jnp.concatenate(
        [edge_index[1].reshape(NTILES, E // NTILES), pad_dst], axis=1
    ).reshape(NTILES, CPT, CH)
    zeros_acc = jnp.zeros((N_ACC, D), jnp.float32)
    zeros_hist = jnp.zeros((HCOL * N,), jnp.float32)
    dst_flat = edge_index[1].reshape(NTILES, EPT)

    degp = _sc_degree(dst_flat, zeros_hist).reshape(NTILES * HCOL, N)
    xw1 = _tc_mm1(x, W1)
    dinvc = _tc_fold(degp).reshape(N, 1)
    hp1, dinv = _tc_scale1(xw1, dinvc)
    agg1 = _sc_gather_scatter(hp1, src2, dst2, zeros_acc)
    hp2 = _tc_mid(agg1, hp1, dinv, b1.reshape(1, D), W2)
    agg2 = _sc_gather_scatter(hp2, src2, dst2, zeros_acc)
    return _tc_last(agg2, hp2, dinv, b2.reshape(1, D), W_out, b_out.reshape(1, D))


# speedup vs baseline: 1.0057x; 1.0057x over previous
import functools

import jax
import jax.numpy as jnp
from jax import lax
from jax.experimental import pallas as pl
from jax.experimental.pallas import tpu as pltpu
from jax.experimental.pallas import tpu_sc as plsc

N = 10000
D = 128
E = 320000

NTILES = 32
NSUB = 16
CH = 128
CPT = 80
HALF = CPT // 2
E_PAD = NTILES * CPT * CH
N_ACC = N + 8

ROWS_PT = 624
TAIL_OFF = ROWS_PT * NSUB

BM = 1000
GRID = N // BM

_mesh = plsc.VectorSubcoreMesh(core_axis_name="c", subcore_axis_name="s")


def _acc_init(zeros_hbm, acc_sh, s):
    pltpu.sync_copy(
        zeros_hbm.at[pl.ds(s * ROWS_PT, ROWS_PT)],
        acc_sh.at[pl.ds(s * ROWS_PT, ROWS_PT)],
    )

    @pl.when(s == 0)
    def _():
        pltpu.sync_copy(
            zeros_hbm.at[pl.ds(TAIL_OFF, N_ACC - TAIL_OFF)],
            acc_sh.at[pl.ds(TAIL_OFF, N_ACC - TAIL_OFF)],
        )


def _acc_writeout(acc_sh, out_hbm, c, s):
    pltpu.sync_copy(
        acc_sh.at[pl.ds(s * ROWS_PT, ROWS_PT)],
        out_hbm.at[c, pl.ds(s * ROWS_PT, ROWS_PT)],
    )

    @pl.when(s == 0)
    def _():
        pltpu.sync_copy(
            acc_sh.at[pl.ds(TAIL_OFF, N - TAIL_OFF)],
            out_hbm.at[c, pl.ds(TAIL_OFF, N - TAIL_OFF)],
        )


EPT = E // NTILES
HCOL = 8
NVEC = EPT // 16


def _sc_degree(dst_flat, zeros_hist):
    @functools.partial(
        pl.kernel,
        out_type=jax.ShapeDtypeStruct((NTILES * HCOL * N,), jnp.float32),
        mesh=_mesh,
        scratch_types=[
            pltpu.VMEM((EPT,), jnp.int32),
            pltpu.VMEM((HCOL * N,), jnp.float32),
        ],
        compiler_params=pltpu.CompilerParams(needs_layout_passes=False),
    )
    def deg_kernel(dst_hbm, zeros_hbm, out_hbm, dst_t, hist):
        c = lax.axis_index("c")
        s = lax.axis_index("s")
        t = c * NSUB + s
        pltpu.sync_copy(zeros_hbm, hist)
        pltpu.sync_copy(dst_hbm.at[t], dst_t)

        lane = lax.iota(jnp.int32, 16)
        base = lax.bitwise_and(lane, 7) * N
        mlo = lane < 8
        mhi = lane >= 8
        ones = jnp.full((16,), 1.0, jnp.float32)

        @pl.loop(0, NVEC)
        def _(i):
            addr = base + dst_t[pl.ds(i * 16, 16)]
            plsc.addupdate_scatter(hist, [addr], ones, mask=mlo)
            plsc.addupdate_scatter(hist, [addr], ones, mask=mhi)

        pltpu.sync_copy(hist, out_hbm.at[pl.ds(t * HCOL * N, HCOL * N)])

    return deg_kernel(dst_flat, zeros_hist)


def _sc_gather_scatter(hp, src2, dst2, zeros_acc):
    @functools.partial(
        pl.kernel,
        out_type=jax.ShapeDtypeStruct((2, N, D), jnp.float32),
        mesh=_mesh,
        scratch_types=[
            pltpu.VMEM((HALF, CH), jnp.int32),
            pltpu.VMEM((HALF, CH), jnp.int32),
            pltpu.VMEM((CH, D), jnp.float32),
            pltpu.VMEM((CH, D), jnp.float32),
            pltpu.VMEM_SHARED((N_ACC, D), jnp.float32),
            pltpu.SemaphoreType.DMA,
            pltpu.SemaphoreType.DMA,
            pltpu.SemaphoreType.DMA,
            pltpu.SemaphoreType.DMA,
        ],
    )
    def gs_kernel(hp_hbm, src_hbm, dst_hbm, zeros_hbm, out_hbm,
                  src_t, dst_t, rows0, rows1, acc_sh, sg0, sg1, ss0, ss1):
        c = lax.axis_index("c")
        s = lax.axis_index("s")
        t = c * NSUB + s
        _acc_init(zeros_hbm, acc_sh, s)
        plsc.subcore_barrier()

        for h in range(CPT // HALF):
            pltpu.sync_copy(src_hbm.at[t, pl.ds(h * HALF, HALF)], src_t)
            pltpu.sync_copy(dst_hbm.at[t, pl.ds(h * HALF, HALF)], dst_t)

            @pl.loop(0, HALF // 2)
            def _(g):
                i0 = 2 * g
                i1 = 2 * g + 1

                @pl.when(g > 0)
                def _():
                    pltpu.make_async_copy(rows0, acc_sh.at[dst_t.at[i0]], ss0).wait()
                    pltpu.make_async_copy(rows1, acc_sh.at[dst_t.at[i1]], ss1).wait()

                g0 = pltpu.async_copy(hp_hbm.at[src_t.at[i0]], rows0, sg0)
                g1 = pltpu.async_copy(hp_hbm.at[src_t.at[i1]], rows1, sg1)
                g0.wait()
                pltpu.async_copy(rows0, acc_sh.at[dst_t.at[i0]], ss0, add=True)
                g1.wait()
                pltpu.async_copy(rows1, acc_sh.at[dst_t.at[i1]], ss1, add=True)

            pltpu.make_async_copy(rows0, acc_sh.at[dst_t.at[0]], ss0).wait()
            pltpu.make_async_copy(rows1, acc_sh.at[dst_t.at[0]], ss1).wait()

        plsc.subcore_barrier()
        _acc_writeout(acc_sh, out_hbm, c, s)

    return gs_kernel(hp, src2, dst2, zeros_acc)


_DN = (((1,), (0,)), ((), ()))


def _matmul(a_ref, w_ref):
    return lax.dot_general(a_ref, w_ref, _DN,
                           preferred_element_type=jnp.float32,
                           precision=lax.Precision.HIGHEST)


def _tc_fold(degp):
    def body(deg_ref, out_ref):
        out_ref[...] = lax.rsqrt(
            jnp.sum(deg_ref[...], axis=0, keepdims=True) + 1.0
        )

    return pl.pallas_call(
        body,
        in_specs=[pl.BlockSpec((NTILES * HCOL, N), lambda: (0, 0))],
        out_specs=pl.BlockSpec((1, N), lambda: (0, 0)),
        out_shape=jax.ShapeDtypeStruct((1, N), jnp.float32),
    )(degp)


def _tc_scale1(xw1, dinvc):
    def body(xw_ref, dinv_ref, hp_ref):
        hp_ref[...] = xw_ref[...] * jnp.broadcast_to(dinv_ref[...], (BM, D))

    return pl.pallas_call(
        body,
        grid=(GRID,),
        in_specs=[
            pl.BlockSpec((BM, D), lambda i: (i, 0)),
            pl.BlockSpec((BM, 1), lambda i: (i, 0)),
        ],
        out_specs=pl.BlockSpec((BM, D), lambda i: (i, 0)),
        out_shape=jax.ShapeDtypeStruct((N, D), jnp.float32),
    )(xw1, dinvc)


def _tc_mm1(x, W1):
    def body(x_ref, w_ref, out_ref):
        out_ref[...] = _matmul(x_ref[...], w_ref[...])

    return pl.pallas_call(
        body,
        grid=(GRID,),
        in_specs=[
            pl.BlockSpec((BM, D), lambda i: (i, 0)),
            pl.BlockSpec((D, D), lambda i: (0, 0)),
        ],
        out_specs=pl.BlockSpec((BM, D), lambda i: (i, 0)),
        out_shape=jax.ShapeDtypeStruct((N, D), jnp.float32),
    )(x, W1)


def _tc_mid(agg, hp, dinvc, b1, W2):
    def body(agg_ref, hp_ref, dinv_ref, b_ref, w_ref, out_ref):
        dinv = jnp.broadcast_to(dinv_ref[...], (BM, D))
        total = agg_ref[0] + agg_ref[1] + hp_ref[...]
        out1 = jnp.maximum(dinv * total + b_ref[...], 0.0)
        out_ref[...] = dinv * _matmul(out1, w_ref[...])

    return pl.pallas_call(
        body,
        grid=(GRID,),
        in_specs=[
            pl.BlockSpec((2, BM, D), lambda i: (0, i, 0)),
            pl.BlockSpec((BM, D), lambda i: (i, 0)),
            pl.BlockSpec((BM, 1), lambda i: (i, 0)),
            pl.BlockSpec((1, D), lambda i: (0, 0)),
            pl.BlockSpec((D, D), lambda i: (0, 0)),
        ],
        out_specs=pl.BlockSpec((BM, D), lambda i: (i, 0)),
        out_shape=jax.ShapeDtypeStruct((N, D), jnp.float32),
    )(agg, hp, dinvc, b1, W2)


def _tc_last(agg, hp, dinvc, b2, W_out, b_out):
    def body(agg_ref, hp_ref, dinv_ref, b2_ref, w_ref, bo_ref, out_ref):
        dinv = jnp.broadcast_to(dinv_ref[...], (BM, D))
        total = agg_ref[0] + agg_ref[1] + hp_ref[...]
        out2 = dinv * total + b2_ref[...]
        out_ref[...] = _matmul(out2, w_ref[...]) + bo_ref[...]

    return pl.pallas_call(
        body,
        grid=(GRID,),
        in_specs=[
            pl.BlockSpec((2, BM, D), lambda i: (0, i, 0)),
            pl.BlockSpec((BM, D), lambda i: (i, 0)),
            pl.BlockSpec((BM, 1), lambda i: (i, 0)),
            pl.BlockSpec((1, D), lambda i: (0, 0)),
            pl.BlockSpec((D, D), lambda i: (0, 0)),
            pl.BlockSpec((1, D), lambda i: (0, 0)),
        ],
        out_specs=pl.BlockSpec((BM, D), lambda i: (i, 0)),
        out_shape=jax.ShapeDtypeStruct((N, D), jnp.float32),
    )(agg, hp, dinvc, b2, W_out, b_out)


def kernel(x, edge_index, W1, b1, W2, b2, W_out, b_out):
    ppt = CPT * CH - E // NTILES
    pad_src = jnp.broadcast_to(jnp.arange(ppt, dtype=jnp.int32), (NTILES, ppt))
    pad_dst = jnp.full((NTILES, ppt), N, jnp.int32)
    src2 = jnp.concatenate(
        [edge_index[0].reshape(NTILES, E // NTILES), pad_src], axis=1
    ).reshape(NTILES, CPT, CH)
    dst2 = jnp.concatenate(
        [edge_index[1].reshape(NTILES, E // NTILES), pad_dst], axis=1
    ).reshape(NTILES, CPT, CH)
    zeros_acc = jnp.zeros((N_ACC, D), jnp.float32)
    zeros_hist = jnp.zeros((HCOL * N,), jnp.float32)
    dst_flat = edge_index[1].reshape(NTILES, EPT)

    degp = _sc_degree(dst_flat, zeros_hist).reshape(NTILES * HCOL, N)
    xw1 = _tc_mm1(x, W1)
    dinvc = _tc_fold(degp).reshape(N, 1)
    hp1 = _tc_scale1(xw1, dinvc)
    agg1 = _sc_gather_scatter(hp1, src2, dst2, zeros_acc)
    hp2 = _tc_mid(agg1, hp1, dinvc, b1.reshape(1, D), W2)
    agg2 = _sc_gather_scatter(hp2, src2, dst2, zeros_acc)
    return _tc_last(agg2, hp2, dinvc, b2.reshape(1, D), W_out, b_out.reshape(1, D))
